# baseline (speedup 1.0000x reference)
import functools

import jax
import jax.numpy as jnp
from jax.experimental import pallas as pl
from jax.experimental.pallas import tpu as pltpu
from jax.experimental.pallas import tpu_sc as plsc

TOPK = 2


def _logits_kernel(x_ref, gw_ref, gb_ref, o_ref):
    logits = jnp.dot(x_ref[:], gw_ref[:],
                     preferred_element_type=jnp.float32) + gb_ref[0][None, :]
    o_ref[:] = logits.T


def _make_sc_topk(N, E):
    info = plsc.get_sparse_core_info()
    NC, NS, L = info.num_cores, info.num_subcores, info.num_lanes
    NW = NC * NS
    chunk = N // NW
    mesh = plsc.VectorSubcoreMesh(core_axis_name="c", subcore_axis_name="s")

    @functools.partial(
        pl.kernel, mesh=mesh,
        out_type=jax.ShapeDtypeStruct((E, N), jnp.float32),
        scratch_types=[
            pltpu.VMEM((E, chunk), jnp.float32),
            pltpu.VMEM((E, chunk), jnp.float32),
        ],
    )
    def topk_sc(logits_hbm, w_hbm, l_v, w_v):
        wid = jax.lax.axis_index("s") * NC + jax.lax.axis_index("c")
        base = wid * chunk
        pltpu.sync_copy(logits_hbm.at[:, pl.ds(base, chunk)], l_v)
        neg_inf = jnp.full((L,), -jnp.inf, jnp.float32)
        for g in range(chunk // L):
            sl = pl.ds(g * L, L)
            vs = [l_v[e, sl] for e in range(E)]
            m1 = vs[0]
            i1 = jnp.zeros((L,), jnp.int32)
            for e in range(1, E):
                cond = vs[e] > m1
                m1 = jnp.where(cond, vs[e], m1)
                i1 = jnp.where(cond, jnp.full((L,), e, jnp.int32), i1)
            m2 = neg_inf
            i2 = jnp.zeros((L,), jnp.int32)
            for e in range(E):
                cand = jnp.where(i1 == e, neg_inf, vs[e])
                cond = cand > m2
                m2 = jnp.where(cond, cand, m2)
                i2 = jnp.where(cond, jnp.full((L,), e, jnp.int32), i2)
            b = jnp.exp(m2 - m1)
            w1 = 1.0 / (1.0 + b)
            w2 = 1.0 - w1
            zero = jnp.zeros((L,), jnp.float32)
            for e in range(E):
                w_v[e, sl] = jnp.where(i1 == e, w1,
                                       jnp.where(i2 == e, w2, zero))
        pltpu.sync_copy(w_v, w_hbm.at[:, pl.ds(base, chunk)])

    return topk_sc


def _moe_kernel(wt_ref, d_ref, ew_ref, eb_ref, o_ref):
    E = wt_ref.shape[0]
    w = wt_ref[:].T
    delta = d_ref[:].astype(jnp.bfloat16)
    acc = jnp.dot(w, eb_ref[:], preferred_element_type=jnp.float32)
    for e in range(E):
        acc = acc + w[:, e:e + 1] * jnp.dot(
            delta, ew_ref[e].astype(jnp.bfloat16),
            preferred_element_type=jnp.float32)
    o_ref[:] = acc


def kernel(input_feat, delta, gate_W, gate_b, expert_W, expert_b):
    B, T, D = input_feat.shape
    E = gate_W.shape[1]
    N = B * T
    x = input_feat.reshape(N, D)
    d = delta.reshape(N, D)
    gb = gate_b.reshape(1, E)

    BLK = 1024
    grid = (N // BLK,)

    LBLK = 4096
    logits_t = pl.pallas_call(
        _logits_kernel,
        grid=(N // LBLK,),
        in_specs=[
            pl.BlockSpec((LBLK, D), lambda i: (i, 0)),
            pl.BlockSpec((D, E), lambda i: (0, 0)),
            pl.BlockSpec((1, E), lambda i: (0, 0)),
        ],
        out_specs=pl.BlockSpec((E, LBLK), lambda i: (0, i)),
        out_shape=jax.ShapeDtypeStruct((E, N), jnp.float32),
    )(x, gate_W, gb)

    w_t = _make_sc_topk(N, E)(logits_t)

    out = pl.pallas_call(
        _moe_kernel,
        grid=grid,
        in_specs=[
            pl.BlockSpec((E, BLK), lambda i: (0, i)),
            pl.BlockSpec((BLK, D), lambda i: (i, 0)),
            pl.BlockSpec((E, D, D), lambda i: (0, 0, 0)),
            pl.BlockSpec((E, D), lambda i: (0, 0)),
        ],
        out_specs=pl.BlockSpec((BLK, D), lambda i: (i, 0)),
        out_shape=jax.ShapeDtypeStruct((N, D), jnp.float32),
    )(w_t, d, expert_W, expert_b)
    return out.reshape(B, T, D)


# speedup vs baseline: 1.0048x; 1.0048x over previous
import functools

import jax
import jax.numpy as jnp
from jax.experimental import pallas as pl
from jax.experimental.pallas import tpu as pltpu
from jax.experimental.pallas import tpu_sc as plsc

TOPK = 2


def _logits_kernel(x_ref, gw_ref, gb_ref, o_ref):
    logits = jnp.dot(x_ref[:], gw_ref[:],
                     preferred_element_type=jnp.float32) + gb_ref[0][None, :]
    o_ref[:] = logits.T


def _make_sc_topk(N, E):
    info = plsc.get_sparse_core_info()
    NC, NS, L = info.num_cores, info.num_subcores, info.num_lanes
    NW = NC * NS
    chunk = N // NW
    mesh = plsc.VectorSubcoreMesh(core_axis_name="c", subcore_axis_name="s")

    @functools.partial(
        pl.kernel, mesh=mesh,
        out_type=jax.ShapeDtypeStruct((E, N), jnp.float32),
        scratch_types=[
            pltpu.VMEM((E, chunk), jnp.float32),
            pltpu.VMEM((E, chunk), jnp.float32),
        ],
    )
    def topk_sc(logits_hbm, w_hbm, l_v, w_v):
        wid = jax.lax.axis_index("s") * NC + jax.lax.axis_index("c")
        base = wid * chunk
        pltpu.sync_copy(logits_hbm.at[:, pl.ds(base, chunk)], l_v)
        neg_inf = jnp.full((L,), -jnp.inf, jnp.float32)
        for g in range(chunk // L):
            sl = pl.ds(g * L, L)
            vs = [l_v[e, sl] for e in range(E)]
            m1 = vs[0]
            i1 = jnp.zeros((L,), jnp.int32)
            for e in range(1, E):
                cond = vs[e] > m1
                m1 = jnp.where(cond, vs[e], m1)
                i1 = jnp.where(cond, jnp.full((L,), e, jnp.int32), i1)
            m2 = neg_inf
            i2 = jnp.zeros((L,), jnp.int32)
            for e in range(E):
                cand = jnp.where(i1 == e, neg_inf, vs[e])
                cond = cand > m2
                m2 = jnp.where(cond, cand, m2)
                i2 = jnp.where(cond, jnp.full((L,), e, jnp.int32), i2)
            b = jnp.exp(m2 - m1)
            w1 = 1.0 / (1.0 + b)
            w2 = 1.0 - w1
            zero = jnp.zeros((L,), jnp.float32)
            for e in range(E):
                w_v[e, sl] = jnp.where(i1 == e, w1,
                                       jnp.where(i2 == e, w2, zero))
        pltpu.sync_copy(w_v, w_hbm.at[:, pl.ds(base, chunk)])

    return topk_sc


def _moe_kernel(wt_ref, d_ref, ew_ref, eb_ref, o_ref):
    E = wt_ref.shape[0]
    w = wt_ref[:].T
    delta = d_ref[:].astype(jnp.bfloat16)
    acc = jnp.dot(w, eb_ref[:], preferred_element_type=jnp.float32)
    for e in range(E):
        acc = acc + w[:, e:e + 1] * jnp.dot(
            delta, ew_ref[e].astype(jnp.bfloat16),
            preferred_element_type=jnp.float32)
    o_ref[:] = acc


def kernel(input_feat, delta, gate_W, gate_b, expert_W, expert_b):
    B, T, D = input_feat.shape
    E = gate_W.shape[1]
    N = B * T
    x = input_feat.reshape(N, D)
    d = delta.reshape(N, D)
    gb = gate_b.reshape(1, E)

    BLK = 1024
    grid = (N // BLK,)

    LBLK = 2048
    logits_t = pl.pallas_call(
        _logits_kernel,
        grid=(N // LBLK,),
        in_specs=[
            pl.BlockSpec((LBLK, D), lambda i: (i, 0)),
            pl.BlockSpec((D, E), lambda i: (0, 0)),
            pl.BlockSpec((1, E), lambda i: (0, 0)),
        ],
        out_specs=pl.BlockSpec((E, LBLK), lambda i: (0, i)),
        out_shape=jax.ShapeDtypeStruct((E, N), jnp.float32),
    )(x, gate_W, gb)

    w_t = _make_sc_topk(N, E)(logits_t)

    out = pl.pallas_call(
        _moe_kernel,
        grid=grid,
        in_specs=[
            pl.BlockSpec((E, BLK), lambda i: (0, i)),
            pl.BlockSpec((BLK, D), lambda i: (i, 0)),
            pl.BlockSpec((E, D, D), lambda i: (0, 0, 0)),
            pl.BlockSpec((E, D), lambda i: (0, 0)),
        ],
        out_specs=pl.BlockSpec((BLK, D), lambda i: (i, 0)),
        out_shape=jax.ShapeDtypeStruct((N, D), jnp.float32),
    )(w_t, d, expert_W, expert_b)
    return out.reshape(B, T, D)
